# Initial kernel scaffold; baseline (speedup 1.0000x reference)
import jax
import jax.numpy as jnp
from jax.experimental import pallas as pl


def kernel(x, weights, anchors, detector_to_output):
    raise NotImplementedError("write your pallas kernel here")


# speedup vs baseline: 491.1039x; 491.1039x over previous
import functools

import jax
import jax.numpy as jnp
from jax import lax
from jax.experimental import pallas as pl
from jax.experimental.pallas import tpu as pltpu
from jax.experimental.pallas import tpu_sc as plsc

B = 1024
NI = 1024
NO = 1024
D = 16384
A = 8
C = 256

NC = 2
NS = 16
NW = NC * NS
L = 16
D_PER_TILE = D // NW

KM = 8
KD = 8


def _mt_build_body(anc_hbm, mt_hbm, anc_v, mt_v):
    cid = lax.axis_index("c")
    sid = lax.axis_index("s")
    wid = sid * NC + cid
    tile_base = wid * D_PER_TILE

    def zrow(r, carry):
        for cc in range(NI // L):
            mt_v[r, pl.ds(cc * L, L)] = jnp.zeros((L,), jnp.float32)
        return carry

    lax.fori_loop(0, KM, zrow, 0)

    def chunk(ch, carry):
        dbase = tile_base + ch * KM
        pltpu.sync_copy(anc_hbm.at[pl.ds(dbase * A, KM * A)], anc_v)
        for g in range(KM * A // L):
            pvec = g * L + lax.iota(jnp.int32, 16)
            row = lax.shift_right_logical(pvec, 3)
            aidx = jnp.bitwise_and(pvec, 7)
            val = jnp.left_shift(jnp.ones((L,), jnp.int32), aidx).astype(jnp.float32)
            col = anc_v[pl.ds(g * L, L)]
            plsc.addupdate_scatter(mt_v, [row, col], val)
        pltpu.sync_copy(mt_v, mt_hbm.at[pl.ds(dbase, KM)])
        for g in range(KM * A // L):
            pvec = g * L + lax.iota(jnp.int32, 16)
            row = lax.shift_right_logical(pvec, 3)
            col = anc_v[pl.ds(g * L, L)]
            plsc.store_scatter(mt_v, [row, col], jnp.zeros((L,), jnp.float32))
        return carry

    lax.fori_loop(0, D_PER_TILE // KM, chunk, 0)


def _mm_body(mt_ref, x_ref, chan_ref):
    bits = (x_ref[...] > 0.0).astype(jnp.bfloat16)
    mtb = mt_ref[...].astype(jnp.bfloat16)
    acc = lax.dot_general(mtb, bits, (((1,), (1,)), ((), ())),
                          preferred_element_type=jnp.float32)
    chan_ref[...] = acc.astype(jnp.int32)


_DB = 2048


def _channel_mm(mt, x):
    return pl.pallas_call(
        _mm_body,
        grid=(D // _DB,),
        in_specs=[
            pl.BlockSpec((_DB, NI), lambda i: (i, 0)),
            pl.BlockSpec((B, NI), lambda i: (0, 0)),
        ],
        out_specs=pl.BlockSpec((_DB, B), lambda i: (i, 0)),
        out_shape=jax.ShapeDtypeStruct((D, B), jnp.int32),
    )(mt, x)


def _lut_body(chan_hbm, w_hbm, dto_hbm, out_hbm,
              chan_v, w_v, dto_v, fired_v, stage_v, acc):
    cid = lax.axis_index("c")
    sid = lax.axis_index("s")
    wid = sid * NC + cid
    rows = NO // NS
    srows = rows // 4

    def zrow(r, carry):
        for cc in range(B // L):
            stage_v[r, pl.ds(cc * L, L)] = jnp.zeros((L,), jnp.float32)
        return carry

    lax.fori_loop(0, srows, zrow, 0)
    for p in range(4):
        pltpu.sync_copy(stage_v, acc.at[pl.ds(sid * rows + p * srows, srows)])
    plsc.subcore_barrier()

    def chunk(ch, carry):
        dbase = wid * D_PER_TILE + ch * KD
        pltpu.sync_copy(chan_hbm.at[pl.ds(dbase, KD)], chan_v)
        pltpu.sync_copy(w_hbm.at[pl.ds(dbase, KD)], w_v)
        pltpu.sync_copy(dto_hbm.at[pl.ds(dbase, KD)], dto_v)
        for j in range(KD):
            rowj = jnp.full((L,), j, jnp.int32)
            for cc in range(B // L):
                idx = chan_v[j, pl.ds(cc * L, L)]
                fired_v[j, pl.ds(cc * L, L)] = plsc.load_gather(w_v, [rowj, idx])
        pltpu.sync_copy(fired_v, acc.at[dto_v], add=True)
        return carry

    lax.fori_loop(0, D_PER_TILE // KD, chunk, 0)
    plsc.subcore_barrier()
    for p in range(4):
        pltpu.sync_copy(acc.at[pl.ds(sid * rows + p * srows, srows)], stage_v)
        pltpu.sync_copy(stage_v, out_hbm.at[cid, pl.ds(sid * rows + p * srows, srows)])


def _comb_body(p_ref, o_ref):
    s = p_ref[0] + p_ref[1]
    o_ref[...] = s.T


def _combine(partial):
    return pl.pallas_call(
        _comb_body,
        out_shape=jax.ShapeDtypeStruct((B, NO), jnp.float32),
    )(partial)


@functools.lru_cache(maxsize=1)
def _sc_kernels():
    mesh = plsc.VectorSubcoreMesh(core_axis_name="c", subcore_axis_name="s",
                                  num_cores=NC, num_subcores=NS)
    sc_params = pltpu.CompilerParams(use_tc_tiling_on_sc=False,
                                     needs_layout_passes=False)
    mt_build = pl.kernel(
        _mt_build_body,
        out_type=jax.ShapeDtypeStruct((D, NI), jnp.float32),
        mesh=mesh,
        compiler_params=sc_params,
        scratch_types=[
            pltpu.VMEM((KM * A,), jnp.int32),
            pltpu.VMEM((KM, NI), jnp.float32),
        ],
    )
    lut = pl.kernel(
        _lut_body,
        out_type=jax.ShapeDtypeStruct((NC, NO, B), jnp.float32),
        mesh=mesh,
        compiler_params=sc_params,
        scratch_types=[
            pltpu.VMEM((KD, B), jnp.int32),
            pltpu.VMEM((KD, C), jnp.float32),
            pltpu.VMEM((KD,), jnp.int32),
            pltpu.VMEM((KD, B), jnp.float32),
            pltpu.VMEM((NO // NS // 4, B), jnp.float32),
            pltpu.VMEM_SHARED((NO, B), jnp.float32),
        ],
    )
    return mt_build, lut


def kernel(x, weights, anchors, detector_to_output):
    mt_build, lut = _sc_kernels()
    anc_flat = anchors.reshape(-1).astype(jnp.int32)
    dto = detector_to_output.astype(jnp.int32)
    mt = mt_build(anc_flat)
    chan = _channel_mm(mt, x)
    partial = lut(chan, weights, dto)
    return _combine(partial)


# speedup vs baseline: 626.5371x; 1.2758x over previous
import functools

import jax
import jax.numpy as jnp
from jax import lax
from jax.experimental import pallas as pl
from jax.experimental.pallas import tpu as pltpu
from jax.experimental.pallas import tpu_sc as plsc

B = 1024
NI = 1024
NO = 1024
D = 16384
A = 8
C = 256

NC = 2
NS = 16
NW = NC * NS
L = 16
D_PER_TILE = D // NW

KM = 8
KD = 8


def _mt_build_body(anc_hbm, mt_hbm, anc_v, mt_v, sem0, sem1):
    cid = lax.axis_index("c")
    sid = lax.axis_index("s")
    wid = sid * NC + cid
    tile_base = wid * D_PER_TILE
    sems = (sem0, sem1)
    nchunk = D_PER_TILE // KM

    pltpu.sync_copy(anc_hbm.at[pl.ds(tile_base * A, D_PER_TILE * A)], anc_v)

    def zrow(r, carry):
        for cc in range(NI // L):
            mt_v[0, r, pl.ds(cc * L, L)] = jnp.zeros((L,), jnp.float32)
            mt_v[1, r, pl.ds(cc * L, L)] = jnp.zeros((L,), jnp.float32)
        return carry

    lax.fori_loop(0, KM, zrow, 0)

    def scatter_chunk(ch, b, undo):
        aoff = ch * (KM * A)
        for g in range(KM * A // L):
            pvec = g * L + lax.iota(jnp.int32, 16)
            row = lax.shift_right_logical(pvec, 3)
            col = anc_v[pl.ds(aoff + g * L, L)]
            if undo:
                plsc.store_scatter(mt_v.at[b], [row, col],
                                   jnp.zeros((L,), jnp.float32))
            else:
                aidx = jnp.bitwise_and(pvec, 7)
                val = jnp.left_shift(jnp.ones((L,), jnp.int32),
                                     aidx).astype(jnp.float32)
                plsc.addupdate_scatter(mt_v.at[b], [row, col], val)

    def outer(g, carry):
        for b in range(2):
            ch = 2 * g + b

            @pl.when(g > 0)
            def _():
                pltpu.make_async_copy(
                    mt_v.at[b], mt_hbm.at[pl.ds(tile_base, KM)], sems[b]).wait()
                scatter_chunk(ch - 2, b, undo=True)

            scatter_chunk(ch, b, undo=False)
            pltpu.async_copy(mt_v.at[b],
                             mt_hbm.at[pl.ds(tile_base + ch * KM, KM)], sems[b])
        return carry

    lax.fori_loop(0, nchunk // 2, outer, 0)
    for b in range(2):
        pltpu.make_async_copy(
            mt_v.at[b], mt_hbm.at[pl.ds(tile_base, KM)], sems[b]).wait()


def _mm_body(mt_ref, x_ref, chan_ref):
    bits = (x_ref[...] > 0.0).astype(jnp.bfloat16)
    mtb = mt_ref[...].astype(jnp.bfloat16)
    acc = lax.dot_general(mtb, bits, (((1,), (1,)), ((), ())),
                          preferred_element_type=jnp.float32)
    chan_ref[...] = acc.astype(jnp.int32)


_DB = 2048


def _channel_mm(mt, x):
    return pl.pallas_call(
        _mm_body,
        grid=(D // _DB,),
        in_specs=[
            pl.BlockSpec((_DB, NI), lambda i: (i, 0)),
            pl.BlockSpec((B, NI), lambda i: (0, 0)),
        ],
        out_specs=pl.BlockSpec((_DB, B), lambda i: (i, 0)),
        out_shape=jax.ShapeDtypeStruct((D, B), jnp.int32),
    )(mt, x)


def _lut_body(chan_hbm, w_hbm, dto2d_hbm, out_hbm,
              chan_v, w_v, dto_v, fired_v, stage_v, acc,
              sem_in0, sem_in1, sem_sc0, sem_sc1):
    cid = lax.axis_index("c")
    sid = lax.axis_index("s")
    wid = sid * NC + cid
    rows = NO // NS
    srows = rows // 4
    sem_in = (sem_in0, sem_in1)
    sem_sc = (sem_sc0, sem_sc1)
    nchunk = D_PER_TILE // KD

    pltpu.sync_copy(dto2d_hbm.at[pl.ds(wid * nchunk, nchunk)], dto_v)

    def zrow(r, carry):
        for cc in range(B // L):
            stage_v[r, pl.ds(cc * L, L)] = jnp.zeros((L,), jnp.float32)
        return carry

    lax.fori_loop(0, srows, zrow, 0)
    for p in range(4):
        pltpu.sync_copy(stage_v, acc.at[pl.ds(sid * rows + p * srows, srows)])
    plsc.subcore_barrier()

    def issue_in(ch, b):
        dbase = wid * D_PER_TILE + ch * KD
        pltpu.async_copy(chan_hbm.at[pl.ds(dbase, KD)], chan_v.at[b], sem_in[b])
        pltpu.async_copy(w_hbm.at[pl.ds(dbase, KD)], w_v.at[b], sem_in[b])

    def wait_in(b):
        pltpu.make_async_copy(chan_hbm.at[pl.ds(0, KD)], chan_v.at[b],
                              sem_in[b]).wait()
        pltpu.make_async_copy(w_hbm.at[pl.ds(0, KD)], w_v.at[b],
                              sem_in[b]).wait()

    issue_in(0, 0)
    issue_in(1, 1)

    def outer(g, carry):
        for b in range(2):
            ch = 2 * g + b

            @pl.when(g > 0)
            def _():
                pltpu.make_async_copy(fired_v.at[b], acc.at[dto_v.at[ch - 2]],
                                      sem_sc[b]).wait()

            wait_in(b)
            for j in range(KD):
                rowj = jnp.full((L,), j, jnp.int32)
                for cc in range(B // L):
                    idx = chan_v[b, j, pl.ds(cc * L, L)]
                    fired_v[b, j, pl.ds(cc * L, L)] = plsc.load_gather(
                        w_v.at[b], [rowj, idx])
            pltpu.async_copy(fired_v.at[b], acc.at[dto_v.at[ch]], sem_sc[b],
                             add=True)

            @pl.when(ch + 2 < nchunk)
            def _():
                issue_in(ch + 2, b)
        return carry

    lax.fori_loop(0, nchunk // 2, outer, 0)
    for b in range(2):
        pltpu.make_async_copy(fired_v.at[b], acc.at[dto_v.at[0]],
                              sem_sc[b]).wait()
    plsc.subcore_barrier()
    for p in range(4):
        pltpu.sync_copy(acc.at[pl.ds(sid * rows + p * srows, srows)], stage_v)
        pltpu.sync_copy(stage_v, out_hbm.at[cid, pl.ds(sid * rows + p * srows, srows)])


def _comb_body(p_ref, o_ref):
    s = p_ref[0] + p_ref[1]
    o_ref[...] = s.T


def _combine(partial):
    return pl.pallas_call(
        _comb_body,
        out_shape=jax.ShapeDtypeStruct((B, NO), jnp.float32),
    )(partial)


@functools.lru_cache(maxsize=1)
def _sc_kernels():
    mesh = plsc.VectorSubcoreMesh(core_axis_name="c", subcore_axis_name="s",
                                  num_cores=NC, num_subcores=NS)
    sc_params = pltpu.CompilerParams(use_tc_tiling_on_sc=False,
                                     needs_layout_passes=False)
    mt_build = pl.kernel(
        _mt_build_body,
        out_type=jax.ShapeDtypeStruct((D, NI), jnp.float32),
        mesh=mesh,
        compiler_params=sc_params,
        scratch_types=[
            pltpu.VMEM((D_PER_TILE * A,), jnp.int32),
            pltpu.VMEM((2, KM, NI), jnp.float32),
            pltpu.SemaphoreType.DMA,
            pltpu.SemaphoreType.DMA,
        ],
    )
    lut = pl.kernel(
        _lut_body,
        out_type=jax.ShapeDtypeStruct((NC, NO, B), jnp.float32),
        mesh=mesh,
        compiler_params=sc_params,
        scratch_types=[
            pltpu.VMEM((2, KD, B), jnp.int32),
            pltpu.VMEM((2, KD, C), jnp.float32),
            pltpu.VMEM((D_PER_TILE // KD, KD), jnp.int32),
            pltpu.VMEM((2, KD, B), jnp.float32),
            pltpu.VMEM((NO // NS // 4, B), jnp.float32),
            pltpu.VMEM_SHARED((NO, B), jnp.float32),
            pltpu.SemaphoreType.DMA,
            pltpu.SemaphoreType.DMA,
            pltpu.SemaphoreType.DMA,
            pltpu.SemaphoreType.DMA,
        ],
    )
    return mt_build, lut


def kernel(x, weights, anchors, detector_to_output):
    mt_build, lut = _sc_kernels()
    anc_flat = anchors.reshape(-1).astype(jnp.int32)
    dto2d = detector_to_output.astype(jnp.int32).reshape(D // KD, KD)
    mt = mt_build(anc_flat)
    chan = _channel_mm(mt, x)
    partial = lut(chan, weights, dto2d)
    return _combine(partial)


# speedup vs baseline: 1629.5628x; 2.6009x over previous
import functools

import jax
import jax.numpy as jnp
from jax import lax
from jax.experimental import pallas as pl
from jax.experimental.pallas import tpu as pltpu
from jax.experimental.pallas import tpu_sc as plsc

B = 1024
NI = 1024
NO = 1024
D = 16384
A = 8
C = 256

NC = 2
NS = 16
NW = NC * NS
L = 16
D_PER_TILE = D // NW

KM = 8
KD = 8


def _mt_build_body(anc_hbm, mt_hbm, anc_v, mt_v, sem0, sem1):
    cid = lax.axis_index("c")
    sid = lax.axis_index("s")
    wid = sid * NC + cid
    tile_base = wid * D_PER_TILE
    sems = (sem0, sem1)
    nchunk = D_PER_TILE // KM

    pltpu.sync_copy(anc_hbm.at[pl.ds(tile_base * A, D_PER_TILE * A)], anc_v)

    def zrow(r, carry):
        for t in range(NI // 128):
            for cc in range(128 // L):
                mt_v[0, r, t, pl.ds(cc * L, L)] = jnp.zeros((L,), jnp.float32)
                mt_v[1, r, t, pl.ds(cc * L, L)] = jnp.zeros((L,), jnp.float32)
        return carry

    lax.fori_loop(0, KM, zrow, 0)

    def scatter_chunk(ch, b, undo):
        aoff = ch * (KM * A)
        for g in range(KM * A // L):
            pvec = g * L + lax.iota(jnp.int32, 16)
            row = lax.shift_right_logical(pvec, 3)
            col = anc_v[pl.ds(aoff + g * L, L)]
            colhi = lax.shift_right_logical(col, 7)
            collo = jnp.bitwise_and(col, 127)
            if undo:
                plsc.store_scatter(mt_v.at[b], [row, colhi, collo],
                                   jnp.zeros((L,), jnp.float32))
            else:
                aidx = jnp.bitwise_and(pvec, 7)
                val = jnp.left_shift(jnp.ones((L,), jnp.int32),
                                     aidx).astype(jnp.float32)
                plsc.addupdate_scatter(mt_v.at[b], [row, colhi, collo], val)

    def outer(g, carry):
        for b in range(2):
            ch = 2 * g + b

            @pl.when(g > 0)
            def _():
                pltpu.make_async_copy(
                    mt_v.at[b], mt_hbm.at[pl.ds(tile_base, KM)], sems[b]).wait()
                scatter_chunk(ch - 2, b, undo=True)

            scatter_chunk(ch, b, undo=False)
            pltpu.async_copy(mt_v.at[b],
                             mt_hbm.at[pl.ds(tile_base + ch * KM, KM)], sems[b])
        return carry

    lax.fori_loop(0, nchunk // 2, outer, 0)
    for b in range(2):
        pltpu.make_async_copy(
            mt_v.at[b], mt_hbm.at[pl.ds(tile_base, KM)], sems[b]).wait()


def _mm_body(mt_ref, x_ref, chan_ref):
    bits = (x_ref[...] > 0.0).astype(jnp.bfloat16)
    mtb = mt_ref[...].reshape(_DB, NI).astype(jnp.bfloat16)
    acc = lax.dot_general(mtb, bits, (((1,), (1,)), ((), ())),
                          preferred_element_type=jnp.float32)
    chan_ref[...] = acc.astype(jnp.int32).reshape(_DB, 8, 128)


_DB = 1024


def _channel_mm(mt, x):
    return pl.pallas_call(
        _mm_body,
        grid=(D // _DB,),
        in_specs=[
            pl.BlockSpec((_DB, 8, 128), lambda i: (i, 0, 0)),
            pl.BlockSpec((B, NI), lambda i: (0, 0)),
        ],
        out_specs=pl.BlockSpec((_DB, 8, 128), lambda i: (i, 0, 0)),
        out_shape=jax.ShapeDtypeStruct((D, 8, 128), jnp.int32),
    )(mt, x)


def _lut_body(chan_hbm, w_hbm, dto2d_hbm, out_hbm,
              chan_v, w_v, dto_v, fired_v, stage_v, acc,
              sem_in0, sem_in1, sem_sc0, sem_sc1):
    cid = lax.axis_index("c")
    sid = lax.axis_index("s")
    wid = sid * NC + cid
    rows = NO // NS
    srows = rows // 4
    sem_in = (sem_in0, sem_in1)
    sem_sc = (sem_sc0, sem_sc1)
    nchunk = D_PER_TILE // KD

    pltpu.sync_copy(dto2d_hbm.at[pl.ds(wid * nchunk, nchunk)], dto_v)

    def zrow(r, carry):
        for t in range(B // 128):
            for cc in range(128 // L):
                stage_v[r, t, pl.ds(cc * L, L)] = jnp.zeros((L,), jnp.float32)
        return carry

    lax.fori_loop(0, srows, zrow, 0)
    for p in range(4):
        pltpu.sync_copy(stage_v, acc.at[pl.ds(sid * rows + p * srows, srows)])
    plsc.subcore_barrier()

    def issue_in(ch, b):
        dbase = wid * D_PER_TILE + ch * KD
        pltpu.async_copy(chan_hbm.at[pl.ds(dbase, KD)], chan_v.at[b], sem_in[b])
        pltpu.async_copy(w_hbm.at[dbase // 8], w_v.at[b], sem_in[b])

    def wait_in(b):
        pltpu.make_async_copy(chan_hbm.at[pl.ds(0, KD)], chan_v.at[b],
                              sem_in[b]).wait()
        pltpu.make_async_copy(w_hbm.at[0], w_v.at[b], sem_in[b]).wait()

    issue_in(0, 0)
    issue_in(1, 1)

    def outer(g, carry):
        for b in range(2):
            ch = 2 * g + b

            @pl.when(g > 0)
            def _():
                pltpu.make_async_copy(fired_v.at[b], acc.at[dto_v.at[ch - 2]],
                                      sem_sc[b]).wait()

            wait_in(b)
            for j in range(KD):
                rowj = jnp.full((L,), j, jnp.int32)

                @plsc.parallel_loop(0, B // L, 1, unroll=8)
                def _(cc):
                    off = pl.multiple_of((cc % 8) * L, L)
                    idx = chan_v[b, j, cc // 8, pl.ds(off, L)]
                    thi = lax.shift_right_logical(idx, 7)
                    tlo = jnp.bitwise_and(idx, 127)
                    fired_v[b, j, cc // 8, pl.ds(off, L)] = plsc.load_gather(
                        w_v.at[b], [thi, rowj, tlo])

            pltpu.async_copy(fired_v.at[b], acc.at[dto_v.at[ch]], sem_sc[b],
                             add=True)

            @pl.when(ch + 2 < nchunk)
            def _():
                issue_in(ch + 2, b)
        return carry

    lax.fori_loop(0, nchunk // 2, outer, 0)
    for b in range(2):
        pltpu.make_async_copy(fired_v.at[b], acc.at[dto_v.at[0]],
                              sem_sc[b]).wait()
    plsc.subcore_barrier()
    for p in range(4):
        pltpu.sync_copy(acc.at[pl.ds(sid * rows + p * srows, srows)], stage_v)
        pltpu.sync_copy(stage_v, out_hbm.at[cid, pl.ds(sid * rows + p * srows, srows)])


def _comb_body(p_ref, o_ref):
    for j in range(8):
        s = p_ref[0, :, j, :] + p_ref[1, :, j, :]
        o_ref[pl.ds(j * 128, 128), :] = s.T


def _combine(partial):
    return pl.pallas_call(
        _comb_body,
        out_shape=jax.ShapeDtypeStruct((B, NO), jnp.float32),
    )(partial)


@functools.lru_cache(maxsize=1)
def _sc_kernels():
    mesh = plsc.VectorSubcoreMesh(core_axis_name="c", subcore_axis_name="s",
                                  num_cores=NC, num_subcores=NS)
    sc_params = pltpu.CompilerParams(use_tc_tiling_on_sc=False,
                                     needs_layout_passes=False)
    mt_build = pl.kernel(
        _mt_build_body,
        out_type=jax.ShapeDtypeStruct((D, 8, 128), jnp.float32),
        mesh=mesh,
        compiler_params=sc_params,
        scratch_types=[
            pltpu.VMEM((D_PER_TILE * A,), jnp.int32),
            pltpu.VMEM((2, KM, 8, 128), jnp.float32),
            pltpu.SemaphoreType.DMA,
            pltpu.SemaphoreType.DMA,
        ],
    )
    lut = pl.kernel(
        _lut_body,
        out_type=jax.ShapeDtypeStruct((NC, NO, 8, 128), jnp.float32),
        mesh=mesh,
        compiler_params=sc_params,
        scratch_types=[
            pltpu.VMEM((2, KD, 8, 128), jnp.int32),
            pltpu.VMEM((2, 2, 8, 128), jnp.float32),
            pltpu.VMEM((D_PER_TILE // KD, KD), jnp.int32),
            pltpu.VMEM((2, KD, 8, 128), jnp.float32),
            pltpu.VMEM((NO // NS // 4, 8, 128), jnp.float32),
            pltpu.VMEM_SHARED((NO, 8, 128), jnp.float32),
            pltpu.SemaphoreType.DMA,
            pltpu.SemaphoreType.DMA,
            pltpu.SemaphoreType.DMA,
            pltpu.SemaphoreType.DMA,
        ],
    )
    return mt_build, lut


def kernel(x, weights, anchors, detector_to_output):
    mt_build, lut = _sc_kernels()
    anc_flat = anchors.reshape(-1).astype(jnp.int32)
    dto2d = detector_to_output.astype(jnp.int32).reshape(D // KD, KD)
    w4 = jnp.swapaxes(weights.reshape(D // 8, 8, C // 128, 128), 1, 2)
    mt = mt_build(anc_flat)
    chan = _channel_mm(mt, x)
    partial = lut(chan, w4, dto2d)
    return _combine(partial)


# speedup vs baseline: 1698.5149x; 1.0423x over previous
import functools

import jax
import jax.numpy as jnp
from jax import lax
from jax.experimental import pallas as pl
from jax.experimental.pallas import tpu as pltpu
from jax.experimental.pallas import tpu_sc as plsc

B = 1024
NI = 1024
NO = 1024
D = 16384
A = 8
C = 256

NC = 2
NS = 16
NW = NC * NS
L = 16
D_PER_TILE = D // NW

KM = 8
KD = 8


def _mt_build_body(anc_hbm, mt_hbm, anc_v, mt_v, sem0, sem1):
    cid = lax.axis_index("c")
    sid = lax.axis_index("s")
    wid = sid * NC + cid
    tile_base = wid * D_PER_TILE
    sems = (sem0, sem1)
    nchunk = D_PER_TILE // KM

    pltpu.sync_copy(anc_hbm.at[pl.ds(tile_base * A, D_PER_TILE * A)], anc_v)

    def zrow(r, carry):
        for t in range(NI // 128):
            for cc in range(128 // L):
                mt_v[0, r, t, pl.ds(cc * L, L)] = jnp.zeros((L,), jnp.float32)
                mt_v[1, r, t, pl.ds(cc * L, L)] = jnp.zeros((L,), jnp.float32)
        return carry

    lax.fori_loop(0, KM, zrow, 0)

    def scatter_chunk(ch, b, undo):
        aoff = ch * (KM * A)
        for g in range(KM * A // L):
            pvec = g * L + lax.iota(jnp.int32, 16)
            row = lax.shift_right_logical(pvec, 3)
            col = anc_v[pl.ds(aoff + g * L, L)]
            colhi = lax.shift_right_logical(col, 7)
            collo = jnp.bitwise_and(col, 127)
            if undo:
                plsc.store_scatter(mt_v.at[b], [row, colhi, collo],
                                   jnp.zeros((L,), jnp.float32))
            else:
                aidx = jnp.bitwise_and(pvec, 7)
                val = jnp.left_shift(jnp.ones((L,), jnp.int32),
                                     aidx).astype(jnp.float32)
                plsc.addupdate_scatter(mt_v.at[b], [row, colhi, collo], val)

    def outer(g, carry):
        for b in range(2):
            ch = 2 * g + b

            @pl.when(g > 0)
            def _():
                pltpu.make_async_copy(
                    mt_v.at[b], mt_hbm.at[pl.ds(tile_base, KM)], sems[b]).wait()
                scatter_chunk(ch - 2, b, undo=True)

            scatter_chunk(ch, b, undo=False)
            pltpu.async_copy(mt_v.at[b],
                             mt_hbm.at[pl.ds(tile_base + ch * KM, KM)], sems[b])
        return carry

    lax.fori_loop(0, nchunk // 2, outer, 0)
    for b in range(2):
        pltpu.make_async_copy(
            mt_v.at[b], mt_hbm.at[pl.ds(tile_base, KM)], sems[b]).wait()


def _mm_body(mt_ref, xe_ref, xo_ref, chan_ref):
    bits_e = (xe_ref[...] > 0.0).astype(jnp.bfloat16)
    bits_o = (xo_ref[...] > 0.0).astype(jnp.bfloat16)
    mtb = mt_ref[...].reshape(_DB, NI).astype(jnp.bfloat16)
    dn = (((1,), (1,)), ((), ()))
    acc_e = lax.dot_general(mtb, bits_e, dn,
                            preferred_element_type=jnp.float32)
    acc_o = lax.dot_general(mtb, bits_o, dn,
                            preferred_element_type=jnp.float32)
    packed = acc_e.astype(jnp.int32) | (acc_o.astype(jnp.int32) << 16)
    chan_ref[...] = packed.reshape(_DB // 2, 8, 128)


_DB = 2048


def _channel_mm(mt, xe, xo):
    return pl.pallas_call(
        _mm_body,
        grid=(D // _DB,),
        in_specs=[
            pl.BlockSpec((_DB, 8, 128), lambda i: (i, 0, 0)),
            pl.BlockSpec((B // 2, NI), lambda i: (0, 0)),
            pl.BlockSpec((B // 2, NI), lambda i: (0, 0)),
        ],
        out_specs=pl.BlockSpec((_DB // 2, 8, 128), lambda i: (i, 0, 0)),
        out_shape=jax.ShapeDtypeStruct((D // 2, 8, 128), jnp.int32),
    )(mt, xe, xo)


def _lut_body(chan_hbm, w_hbm, dto2d_hbm, out_hbm,
              chan_v, w_v, dto_v, fired_v, stage_v, acc,
              sem_in0, sem_in1, sem_sc0, sem_sc1):
    cid = lax.axis_index("c")
    sid = lax.axis_index("s")
    wid = sid * NC + cid
    rows = NO // NS
    srows = rows // 4
    sem_in = (sem_in0, sem_in1)
    sem_sc = (sem_sc0, sem_sc1)
    nchunk = D_PER_TILE // KD

    pltpu.sync_copy(dto2d_hbm.at[pl.ds(wid * nchunk, nchunk)], dto_v)

    def zrow(r, carry):
        for t in range(B // 128):
            for cc in range(128 // L):
                stage_v[r, t, pl.ds(cc * L, L)] = jnp.zeros((L,), jnp.float32)
        return carry

    lax.fori_loop(0, srows, zrow, 0)
    for p in range(4):
        pltpu.sync_copy(stage_v, acc.at[pl.ds(sid * rows + p * srows, srows)])
    plsc.subcore_barrier()

    def issue_in(ch, b):
        dbase = wid * D_PER_TILE + ch * KD
        pltpu.async_copy(chan_hbm.at[pl.ds(dbase // 2, KD // 2)], chan_v.at[b],
                         sem_in[b])
        pltpu.async_copy(w_hbm.at[pl.ds(dbase * C, KD * C)], w_v.at[b],
                         sem_in[b])

    def wait_in(b):
        pltpu.make_async_copy(chan_hbm.at[pl.ds(0, KD // 2)], chan_v.at[b],
                              sem_in[b]).wait()
        pltpu.make_async_copy(w_hbm.at[pl.ds(0, KD * C)], w_v.at[b],
                              sem_in[b]).wait()

    issue_in(0, 0)
    issue_in(1, 1)

    def outer(g, carry):
        for b in range(2):
            ch = 2 * g + b

            @pl.when(g > 0)
            def _():
                pltpu.make_async_copy(fired_v.at[b], acc.at[dto_v.at[ch - 2]],
                                      sem_sc[b]).wait()

            wait_in(b)
            iota2 = lax.iota(jnp.int32, 16) * 2
            for j in range(KD):
                wrow = w_v.at[b, pl.ds(j * C, C)]
                frow = fired_v.at[b, j]

                @plsc.parallel_loop(0, B // (2 * L), 1, unroll=8)
                def _(cc):
                    off = pl.multiple_of((cc % 8) * L, L)
                    word = chan_v[b, j // 2, (j % 2) * 4 + cc // 8,
                                  pl.ds(off, L)]
                    idx_e = jnp.bitwise_and(word, 0xFFFF)
                    idx_o = lax.shift_right_logical(word, 16)
                    fe = plsc.load_gather(wrow, [idx_e])
                    fo = plsc.load_gather(wrow, [idx_o])
                    pos_e = iota2 + (cc % 4) * 32
                    plsc.store_scatter(frow.at[cc // 4], [pos_e], fe)
                    plsc.store_scatter(frow.at[cc // 4], [pos_e + 1], fo)

            pltpu.async_copy(fired_v.at[b], acc.at[dto_v.at[ch]], sem_sc[b],
                             add=True)

            @pl.when(ch + 2 < nchunk)
            def _():
                issue_in(ch + 2, b)
        return carry

    lax.fori_loop(0, nchunk // 2, outer, 0)
    for b in range(2):
        pltpu.make_async_copy(fired_v.at[b], acc.at[dto_v.at[0]],
                              sem_sc[b]).wait()
    plsc.subcore_barrier()
    for p in range(4):
        pltpu.sync_copy(acc.at[pl.ds(sid * rows + p * srows, srows)], stage_v)
        pltpu.sync_copy(stage_v, out_hbm.at[cid, pl.ds(sid * rows + p * srows, srows)])


def _comb_body(p_ref, o_ref):
    for j in range(8):
        s = p_ref[0, :, j, :] + p_ref[1, :, j, :]
        o_ref[pl.ds(j * 128, 128), :] = s.T


def _combine(partial):
    return pl.pallas_call(
        _comb_body,
        out_shape=jax.ShapeDtypeStruct((B, NO), jnp.float32),
    )(partial)


@functools.lru_cache(maxsize=1)
def _sc_kernels():
    mesh = plsc.VectorSubcoreMesh(core_axis_name="c", subcore_axis_name="s",
                                  num_cores=NC, num_subcores=NS)
    sc_params = pltpu.CompilerParams(use_tc_tiling_on_sc=False,
                                     needs_layout_passes=False)
    mt_build = pl.kernel(
        _mt_build_body,
        out_type=jax.ShapeDtypeStruct((D, 8, 128), jnp.float32),
        mesh=mesh,
        compiler_params=sc_params,
        scratch_types=[
            pltpu.VMEM((D_PER_TILE * A,), jnp.int32),
            pltpu.VMEM((2, KM, 8, 128), jnp.float32),
            pltpu.SemaphoreType.DMA,
            pltpu.SemaphoreType.DMA,
        ],
    )
    lut = pl.kernel(
        _lut_body,
        out_type=jax.ShapeDtypeStruct((NC, NO, 8, 128), jnp.float32),
        mesh=mesh,
        compiler_params=sc_params,
        scratch_types=[
            pltpu.VMEM((2, KD // 2, 8, 128), jnp.int32),
            pltpu.VMEM((2, KD * C), jnp.float32),
            pltpu.VMEM((D_PER_TILE // KD, KD), jnp.int32),
            pltpu.VMEM((2, KD, 8, 128), jnp.float32),
            pltpu.VMEM((NO // NS // 4, 8, 128), jnp.float32),
            pltpu.VMEM_SHARED((NO, 8, 128), jnp.float32),
            pltpu.SemaphoreType.DMA,
            pltpu.SemaphoreType.DMA,
            pltpu.SemaphoreType.DMA,
            pltpu.SemaphoreType.DMA,
        ],
    )
    return mt_build, lut


def kernel(x, weights, anchors, detector_to_output):
    mt_build, lut = _sc_kernels()
    anc_flat = anchors.reshape(-1).astype(jnp.int32)
    dto2d = detector_to_output.astype(jnp.int32).reshape(D // KD, KD)
    w_flat = weights.reshape(-1)
    xe = x[0::2]
    xo = x[1::2]
    mt = mt_build(anc_flat)
    chan = _channel_mm(mt, xe, xo)
    partial = lut(chan, w_flat, dto2d)
    return _combine(partial)


# speedup vs baseline: 1709.7881x; 1.0066x over previous
import functools

import jax
import jax.numpy as jnp
from jax import lax
from jax.experimental import pallas as pl
from jax.experimental.pallas import tpu as pltpu
from jax.experimental.pallas import tpu_sc as plsc

B = 1024
NI = 1024
NO = 1024
D = 16384
A = 8
C = 256

NC = 2
NS = 16
NW = NC * NS
L = 16
D_PER_TILE = D // NW

KM = 8
KD = 8


def _mt_build_body(anc_hbm, mt_hbm, anc_v, mt_v, sem0, sem1):
    cid = lax.axis_index("c")
    sid = lax.axis_index("s")
    wid = sid * NC + cid
    tile_base = wid * D_PER_TILE
    sems = (sem0, sem1)
    nchunk = D_PER_TILE // KM

    pltpu.sync_copy(anc_hbm.at[pl.ds(tile_base * A, D_PER_TILE * A)], anc_v)

    def zrow(r, carry):
        for t in range(NI // 128):
            for cc in range(128 // L):
                mt_v[0, r, t, pl.ds(cc * L, L)] = jnp.zeros((L,), jnp.float32)
                mt_v[1, r, t, pl.ds(cc * L, L)] = jnp.zeros((L,), jnp.float32)
        return carry

    lax.fori_loop(0, KM, zrow, 0)

    def scatter_chunk(ch, b, undo):
        aoff = ch * (KM * A)
        for g in range(KM * A // L):
            pvec = g * L + lax.iota(jnp.int32, 16)
            row = lax.shift_right_logical(pvec, 3)
            col = anc_v[pl.ds(aoff + g * L, L)]
            colhi = lax.shift_right_logical(col, 7)
            collo = jnp.bitwise_and(col, 127)
            if undo:
                plsc.store_scatter(mt_v.at[b], [row, colhi, collo],
                                   jnp.zeros((L,), jnp.float32))
            else:
                aidx = jnp.bitwise_and(pvec, 7)
                val = jnp.left_shift(jnp.ones((L,), jnp.int32),
                                     aidx).astype(jnp.float32)
                plsc.addupdate_scatter(mt_v.at[b], [row, colhi, collo], val)

    def outer(g, carry):
        for b in range(2):
            ch = 2 * g + b

            @pl.when(g > 0)
            def _():
                pltpu.make_async_copy(
                    mt_v.at[b], mt_hbm.at[pl.ds(tile_base, KM)], sems[b]).wait()
                scatter_chunk(ch - 2, b, undo=True)

            scatter_chunk(ch, b, undo=False)
            pltpu.async_copy(mt_v.at[b],
                             mt_hbm.at[pl.ds(tile_base + ch * KM, KM)], sems[b])
        return carry

    lax.fori_loop(0, nchunk // 2, outer, 0)
    for b in range(2):
        pltpu.make_async_copy(
            mt_v.at[b], mt_hbm.at[pl.ds(tile_base, KM)], sems[b]).wait()


def _mm_body(mt_ref, xe_ref, xo_ref, chan_ref):
    bits_e = (xe_ref[...] > 0.0).astype(jnp.bfloat16)
    bits_o = (xo_ref[...] > 0.0).astype(jnp.bfloat16)
    mtb = mt_ref[...].reshape(_DB, NI).astype(jnp.bfloat16)
    dn = (((1,), (1,)), ((), ()))
    acc_e = lax.dot_general(mtb, bits_e, dn,
                            preferred_element_type=jnp.float32)
    acc_o = lax.dot_general(mtb, bits_o, dn,
                            preferred_element_type=jnp.float32)
    packed = acc_e.astype(jnp.int32) | (acc_o.astype(jnp.int32) << 16)
    chan_ref[...] = packed.reshape(_DB // 2, 8, 128)


_DB = 2048


def _channel_mm(mt, xe, xo):
    return pl.pallas_call(
        _mm_body,
        grid=(D // _DB,),
        in_specs=[
            pl.BlockSpec((_DB, 8, 128), lambda i: (i, 0, 0)),
            pl.BlockSpec((B // 2, NI), lambda i: (0, 0)),
            pl.BlockSpec((B // 2, NI), lambda i: (0, 0)),
        ],
        out_specs=pl.BlockSpec((_DB // 2, 8, 128), lambda i: (i, 0, 0)),
        out_shape=jax.ShapeDtypeStruct((D // 2, 8, 128), jnp.int32),
    )(mt, xe, xo)


def _lut_body(chan_hbm, w_hbm, dto2d_hbm, out_hbm,
              chan_v, w_v, dto_v, fired_v, stage_v, acc,
              sem_in0, sem_in1, sem_sc0, sem_sc1):
    cid = lax.axis_index("c")
    sid = lax.axis_index("s")
    wid = sid * NC + cid
    rows = NO // NS
    srows = rows // 4
    sem_in = (sem_in0, sem_in1)
    sem_sc = (sem_sc0, sem_sc1)
    nchunk = D_PER_TILE // KD

    pltpu.sync_copy(dto2d_hbm.at[pl.ds(wid * nchunk, nchunk)], dto_v)

    def zrow(r, carry):
        for t in range(B // 128):
            for cc in range(128 // L):
                stage_v[r, t, pl.ds(cc * L, L)] = jnp.zeros((L,), jnp.float32)
        return carry

    lax.fori_loop(0, srows, zrow, 0)
    for p in range(4):
        pltpu.sync_copy(stage_v, acc.at[pl.ds(sid * rows + p * srows, srows)])
    plsc.subcore_barrier()

    def issue_in(ch, b):
        dbase = wid * D_PER_TILE + ch * KD
        pltpu.async_copy(chan_hbm.at[pl.ds(dbase // 2, KD // 2)], chan_v.at[b],
                         sem_in[b])
        pltpu.async_copy(w_hbm.at[dbase // 8], w_v.at[b], sem_in[b])

    def wait_in(b):
        pltpu.make_async_copy(chan_hbm.at[pl.ds(0, KD // 2)], chan_v.at[b],
                              sem_in[b]).wait()
        pltpu.make_async_copy(w_hbm.at[0], w_v.at[b], sem_in[b]).wait()

    issue_in(0, 0)
    issue_in(1, 1)

    def outer(g, carry):
        for b in range(2):
            ch = 2 * g + b

            @pl.when(g > 0)
            def _():
                pltpu.make_async_copy(fired_v.at[b], acc.at[dto_v.at[ch - 2]],
                                      sem_sc[b]).wait()

            wait_in(b)
            for j in range(KD):
                rowj = jnp.full((L,), j, jnp.int32)
                wref = w_v.at[b]

                @plsc.parallel_loop(0, B // (2 * L), 1, unroll=8)
                def _(cc):
                    off = pl.multiple_of((cc % 8) * L, L)
                    word = chan_v[b, j // 2, (j % 2) * 4 + cc // 8,
                                  pl.ds(off, L)]
                    idx_e = jnp.bitwise_and(word, 0xFFFF)
                    idx_o = lax.shift_right_logical(word, 16)
                    fe = plsc.load_gather(
                        wref, [lax.shift_right_logical(idx_e, 7), rowj,
                               jnp.bitwise_and(idx_e, 127)])
                    fo = plsc.load_gather(
                        wref, [lax.shift_right_logical(idx_o, 7), rowj,
                               jnp.bitwise_and(idx_o, 127)])
                    fired_v[b, j, cc // 8, pl.ds(off, L)] = fe
                    fired_v[b, j, 4 + cc // 8, pl.ds(off, L)] = fo

            pltpu.async_copy(fired_v.at[b], acc.at[dto_v.at[ch]], sem_sc[b],
                             add=True)

            @pl.when(ch + 2 < nchunk)
            def _():
                issue_in(ch + 2, b)
        return carry

    lax.fori_loop(0, nchunk // 2, outer, 0)
    for b in range(2):
        pltpu.make_async_copy(fired_v.at[b], acc.at[dto_v.at[0]],
                              sem_sc[b]).wait()
    plsc.subcore_barrier()
    for p in range(4):
        pltpu.sync_copy(acc.at[pl.ds(sid * rows + p * srows, srows)], stage_v)
        pltpu.sync_copy(stage_v, out_hbm.at[cid, pl.ds(sid * rows + p * srows, srows)])


def _comb_body(p_ref, o_ref):
    for j in range(8):
        s = p_ref[0, :, j, :] + p_ref[1, :, j, :]
        o_ref[pl.ds(j * 128, 128), :] = s.T


def _combine(partial):
    return pl.pallas_call(
        _comb_body,
        out_shape=jax.ShapeDtypeStruct((B, NO), jnp.float32),
    )(partial)


@functools.lru_cache(maxsize=1)
def _sc_kernels():
    mesh = plsc.VectorSubcoreMesh(core_axis_name="c", subcore_axis_name="s",
                                  num_cores=NC, num_subcores=NS)
    sc_params = pltpu.CompilerParams(use_tc_tiling_on_sc=False,
                                     needs_layout_passes=False)
    mt_build = pl.kernel(
        _mt_build_body,
        out_type=jax.ShapeDtypeStruct((D, 8, 128), jnp.float32),
        mesh=mesh,
        compiler_params=sc_params,
        scratch_types=[
            pltpu.VMEM((D_PER_TILE * A,), jnp.int32),
            pltpu.VMEM((2, KM, 8, 128), jnp.float32),
            pltpu.SemaphoreType.DMA,
            pltpu.SemaphoreType.DMA,
        ],
    )
    lut = pl.kernel(
        _lut_body,
        out_type=jax.ShapeDtypeStruct((NC, NO, 8, 128), jnp.float32),
        mesh=mesh,
        compiler_params=sc_params,
        scratch_types=[
            pltpu.VMEM((2, KD // 2, 8, 128), jnp.int32),
            pltpu.VMEM((2, 2, 8, 128), jnp.float32),
            pltpu.VMEM((D_PER_TILE // KD, KD), jnp.int32),
            pltpu.VMEM((2, KD, 8, 128), jnp.float32),
            pltpu.VMEM((NO // NS // 4, 8, 128), jnp.float32),
            pltpu.VMEM_SHARED((NO, 8, 128), jnp.float32),
            pltpu.SemaphoreType.DMA,
            pltpu.SemaphoreType.DMA,
            pltpu.SemaphoreType.DMA,
            pltpu.SemaphoreType.DMA,
        ],
    )
    return mt_build, lut


def kernel(x, weights, anchors, detector_to_output):
    mt_build, lut = _sc_kernels()
    anc_flat = anchors.reshape(-1).astype(jnp.int32)
    dto2d = detector_to_output.astype(jnp.int32).reshape(D // KD, KD)
    w4 = jnp.swapaxes(weights.reshape(D // 8, 8, C // 128, 128), 1, 2)
    xe = x[: B // 2]
    xo = x[B // 2:]
    mt = mt_build(anc_flat)
    chan = _channel_mm(mt, xe, xo)
    partial = lut(chan, w4, dto2d)
    return _combine(partial)
